# baseline (speedup 1.0000x reference)
import functools

import jax
import jax.numpy as jnp
from jax import lax
from jax.experimental import pallas as pl
from jax.experimental.pallas import tpu as pltpu
from jax.experimental.pallas import tpu_sc as plsc

BATCH = 4096
SEQ_LEN = 200
EMBED_DIM = 64

NC = 2
NS = 16
NW = NC * NS
LANES = 16

BT = BATCH // 128
DT = EMBED_DIM // 8
NUNITS = SEQ_LEN
NBUF = 4


def _make_kernel():
  mesh = plsc.VectorSubcoreMesh(
      core_axis_name="c", subcore_axis_name="s",
      num_cores=NC, num_subcores=NS)

  @functools.partial(
      pl.kernel,
      out_type=jax.ShapeDtypeStruct((SEQ_LEN, DT, BT, 8, 128), jnp.float32),
      mesh=mesh,
      scratch_types=[
          pltpu.VMEM((NBUF, 128), jnp.int32),
          pltpu.VMEM((NBUF, 128, EMBED_DIM), jnp.float32),
          pltpu.VMEM((NBUF, DT, 8, 128), jnp.float32),
          pltpu.VMEM((SEQ_LEN, EMBED_DIM), jnp.float32),
          [pltpu.SemaphoreType.DMA] * NBUF,
          [pltpu.SemaphoreType.DMA] * NBUF,
          [pltpu.SemaphoreType.DMA] * NBUF,
      ],
      compiler_params=pltpu.CompilerParams(use_tc_tiling_on_sc=False,
                                           needs_layout_passes=False,
                                           disable_bounds_checks=True),
  )
  def emb_kernel(x5_hbm, table_hbm, pos_hbm, out_hbm,
                 idx_v, rows_v, obuf_v, pos_v, isem, gsem, wsem):
    cid = lax.axis_index("c")
    sid = lax.axis_index("s")
    wid = sid * NC + cid

    iota16 = lax.iota(jnp.int32, 16)
    row_idx = [iota16 + 16 * k for k in range(8)]

    def idx_start(u, slot):
      pltpu.async_copy(x5_hbm.at[u // 8, wid, u % 8], idx_v.at[slot],
                       isem[slot])

    def idx_wait(slot):
      pltpu.make_async_copy(x5_hbm.at[0, 0, 0], idx_v.at[slot],
                            isem[slot]).wait()

    def gather_start(slot):
      pltpu.async_copy(table_hbm.at[idx_v.at[slot]], rows_v.at[slot],
                       gsem[slot])

    def gather_wait(slot):
      pltpu.make_async_copy(table_hbm.at[idx_v.at[slot]], rows_v.at[slot],
                            gsem[slot]).wait()

    def wb_start(u, slot):
      pltpu.async_copy(obuf_v.at[slot], out_hbm.at[u, :, wid], wsem[slot])

    def wb_wait(slot):
      pltpu.make_async_copy(obuf_v.at[slot], out_hbm.at[0, :, 0],
                            wsem[slot]).wait()

    def transpose_unit(u, slot):
      rows = rows_v.at[slot]
      obuf = obuf_v.at[slot]
      u16 = jnp.broadcast_to(u, (16,))

      @pl.loop(0, DT)
      def dt_loop(dt):
        obuf_dt = obuf.at[dt]
        d0 = dt * 8
        for dr in range(8):
          col_d = jnp.broadcast_to(d0 + dr, (16,))
          addend = plsc.load_gather(pos_v, [u16, col_d])
          for k in range(8):
            v = plsc.load_gather(rows, [row_idx[k], col_d])
            obuf_dt[dr, pl.ds(16 * k, 16)] = v + addend

    pltpu.sync_copy(pos_hbm, pos_v)

    for c in range(3):
      idx_start(c, c)
    for c in range(2):
      idx_wait(c)
      gather_start(c)

    @pl.loop(0, NUNITS, step=NBUF)
    def main_loop(u0):
      for b in range(NBUF):
        u = u0 + b

        @pl.when(u >= NBUF)
        def _():
          wb_wait(b)

        @pl.when(u + 3 < NUNITS)
        def _():
          idx_start(u + 3, (b + 3) % NBUF)

        @pl.when(u + 2 < NUNITS)
        def _():
          idx_wait((b + 2) % NBUF)
          gather_start((b + 2) % NBUF)

        gather_wait(b)
        transpose_unit(u, b)
        wb_start(u, b)

    for b in range(NBUF):
      wb_wait(b)

  return emb_kernel


_emb_kernel = _make_kernel()


def kernel(x, word_table, pos_table):
  x5 = (x.astype(jnp.int32).T
        .reshape(SEQ_LEN // 8, 8, BT, 128)
        .transpose(0, 2, 1, 3))
  o5 = _emb_kernel(x5, word_table, pos_table[:SEQ_LEN])
  return o5.transpose(2, 4, 0, 1, 3).reshape(BATCH, SEQ_LEN, EMBED_DIM)


# speedup vs baseline: 1.2003x; 1.2003x over previous
import functools

import jax
import jax.numpy as jnp
from jax import lax
from jax.experimental import pallas as pl
from jax.experimental.pallas import tpu as pltpu
from jax.experimental.pallas import tpu_sc as plsc

BATCH = 4096
SEQ_LEN = 200
EMBED_DIM = 64

NC = 2
NS = 16
NW = NC * NS
LANES = 16

BT = BATCH // 128
DT = EMBED_DIM // 8
NUNITS = SEQ_LEN
NBUF = 4


def _make_kernel():
  mesh = plsc.VectorSubcoreMesh(
      core_axis_name="c", subcore_axis_name="s",
      num_cores=NC, num_subcores=NS)

  @functools.partial(
      pl.kernel,
      out_type=jax.ShapeDtypeStruct((SEQ_LEN, DT, BT, 8, 128), jnp.float32),
      mesh=mesh,
      scratch_types=[
          pltpu.VMEM((NBUF, 128), jnp.int32),
          pltpu.VMEM((NBUF, 128, EMBED_DIM), jnp.float32),
          pltpu.VMEM((128, 65), jnp.float32),
          pltpu.VMEM((NBUF, DT, 8, 128), jnp.float32),
          pltpu.VMEM((SEQ_LEN, EMBED_DIM), jnp.float32),
          [pltpu.SemaphoreType.DMA] * NBUF,
          [pltpu.SemaphoreType.DMA] * NBUF,
          [pltpu.SemaphoreType.DMA] * NBUF,
      ],
      compiler_params=pltpu.CompilerParams(use_tc_tiling_on_sc=False,
                                           needs_layout_passes=False,
                                           disable_bounds_checks=True),
  )
  def emb_kernel(x5_hbm, table_hbm, pos_hbm, out_hbm,
                 idx_v, rows_v, rpad_v, obuf_v, pos_v, isem, gsem, wsem):
    cid = lax.axis_index("c")
    sid = lax.axis_index("s")
    wid = sid * NC + cid

    iota16 = lax.iota(jnp.int32, 16)
    row_idx = [iota16 + 16 * k for k in range(8)]

    def idx_start(u, slot):
      pltpu.async_copy(x5_hbm.at[u // 8, wid, u % 8], idx_v.at[slot],
                       isem[slot])

    def idx_wait(slot):
      pltpu.make_async_copy(x5_hbm.at[0, 0, 0], idx_v.at[slot],
                            isem[slot]).wait()

    def gather_start(slot):
      pltpu.async_copy(table_hbm.at[idx_v.at[slot]], rows_v.at[slot],
                       gsem[slot])

    def gather_wait(slot):
      pltpu.make_async_copy(table_hbm.at[idx_v.at[slot]], rows_v.at[slot],
                            gsem[slot]).wait()

    def wb_start(u, slot):
      pltpu.async_copy(obuf_v.at[slot], out_hbm.at[u, :, wid], wsem[slot])

    def wb_wait(slot):
      pltpu.make_async_copy(obuf_v.at[slot], out_hbm.at[0, :, 0],
                            wsem[slot]).wait()

    def transpose_unit(u, slot):
      rows = rows_v.at[slot]
      obuf = obuf_v.at[slot]

      prow = [pos_v[u, pl.ds(16 * q, 16)] for q in range(4)]

      @pl.loop(0, 128, unroll=2)
      def r_loop(r):
        for q in range(4):
          rpad_v[r, pl.ds(16 * q, 16)] = rows[r, pl.ds(16 * q, 16)] + prow[q]

      @pl.loop(0, DT)
      def dt_loop(dt):
        obuf_dt = obuf.at[dt]
        d0 = dt * 8
        for dr in range(8):
          col_d = jnp.broadcast_to(d0 + dr, (16,))
          for k in range(8):
            obuf_dt[dr, pl.ds(16 * k, 16)] = plsc.load_gather(
                rpad_v, [row_idx[k], col_d])

    pltpu.sync_copy(pos_hbm, pos_v)

    for c in range(3):
      idx_start(c, c)
    for c in range(2):
      idx_wait(c)
      gather_start(c)

    @pl.loop(0, NUNITS, step=NBUF)
    def main_loop(u0):
      for b in range(NBUF):
        u = u0 + b

        @pl.when(u >= NBUF)
        def _():
          wb_wait(b)

        @pl.when(u + 3 < NUNITS)
        def _():
          idx_start(u + 3, (b + 3) % NBUF)

        @pl.when(u + 2 < NUNITS)
        def _():
          idx_wait((b + 2) % NBUF)
          gather_start((b + 2) % NBUF)

        gather_wait(b)
        transpose_unit(u, b)
        wb_start(u, b)

    for b in range(NBUF):
      wb_wait(b)

  return emb_kernel


_emb_kernel = _make_kernel()


def kernel(x, word_table, pos_table):
  x5 = (x.astype(jnp.int32).T
        .reshape(SEQ_LEN // 8, 8, BT, 128)
        .transpose(0, 2, 1, 3))
  o5 = _emb_kernel(x5, word_table, pos_table[:SEQ_LEN])
  return o5.transpose(2, 4, 0, 1, 3).reshape(BATCH, SEQ_LEN, EMBED_DIM)
